# baseline (speedup 1.0000x reference)
import functools

import jax
import jax.numpy as jnp
from jax import lax
from jax.experimental import pallas as pl
from jax.experimental.pallas import tpu as pltpu
from jax.experimental.pallas import tpu_sc as plsc

_B, _T, _EMB = 4096, 200, 64
_BB = 128
_EG = _EMB // 8
_NBUF = 4
_LOOK = 2


def _sc_embed(xT, table, pos):
    info = plsc.get_sparse_core_info()
    nw = info.num_cores * info.num_subcores
    nb = _B // _BB
    assert nb == nw

    mesh = plsc.VectorSubcoreMesh(core_axis_name="c", subcore_axis_name="s")

    @functools.partial(
        pl.kernel,
        out_type=jax.ShapeDtypeStruct((_T, _EG, nb, 8 * _BB), jnp.float32),
        mesh=mesh,
        scratch_types=[
            pltpu.VMEM((_T, _BB), jnp.int32),
            pltpu.VMEM((_T, _EMB), jnp.float32),
            pltpu.VMEM((_NBUF, _BB, _EMB), jnp.float32),
            pltpu.VMEM((_NBUF, _EMB * _BB), jnp.float32),
        ]
        + [pltpu.SemaphoreType.DMA] * (2 * _NBUF),
        compiler_params=pltpu.CompilerParams(
            use_tc_tiling_on_sc=False,
            needs_layout_passes=False,
            disable_bounds_checks=True
        ),
    )
    def k(xT_hbm, table_hbm, pos_hbm, out_hbm, idx_v, pos_v, rows_v, tb_v, *sems):
        sem_g = sems[:_NBUF]
        sem_o = sems[_NBUF:]
        wid = lax.axis_index("s") * info.num_cores + lax.axis_index("c")

        pltpu.sync_copy(xT_hbm.at[:, pl.ds(wid * _BB, _BB)], idx_v)
        pltpu.sync_copy(pos_hbm, pos_v)

        iota = lax.iota(jnp.int32, 16)

        def gather_start(t, b):
            pltpu.async_copy(table_hbm.at[idx_v.at[t]], rows_v.at[b], sem_g[b])

        def gather_wait(t, b):
            pltpu.make_async_copy(
                table_hbm.at[idx_v.at[t]], rows_v.at[b], sem_g[b]
            ).wait()

        def out_start(t, b):
            for e in range(_EG):
                pltpu.async_copy(
                    tb_v.at[b, pl.ds(e * 8 * _BB, 8 * _BB)],
                    out_hbm.at[t, e, wid],
                    sem_o[b],
                )

        def out_wait(t, b):
            for e in range(_EG):
                pltpu.make_async_copy(
                    tb_v.at[b, pl.ds(e * 8 * _BB, 8 * _BB)],
                    out_hbm.at[t, e, wid],
                    sem_o[b],
                ).wait()

        for t in range(_LOOK):
            gather_start(t, t % _NBUF)

        @pl.loop(0, _T // _NBUF)
        def _grp(g):
            for b in range(_NBUF):
                t = g * _NBUF + b
                tt = t + _LOOK
                b2 = (b + _LOOK) % _NBUF

                @pl.when(tt < _T)
                def _issue():
                    @pl.when(tt >= _NBUF)
                    def _drain():
                        out_wait(tt - _NBUF, b2)

                    gather_start(tt, b2)

                gather_wait(t, b)

                rb = rows_v.at[b]
                tbb = tb_v.at[b]
                t_splat = jnp.full((16,), t, jnp.int32)
                jidx = [iota + (jg * 16) for jg in range(_BB // 16)]

                @plsc.parallel_loop(0, _EMB, unroll=2)
                def _erow(e):
                    e_splat = jnp.full((16,), e, jnp.int32)
                    pvec = plsc.load_gather(pos_v, [t_splat, e_splat])
                    ebase = e * _BB
                    for jg in range(_BB // 16):
                        v = plsc.load_gather(rb, [jidx[jg], e_splat]) + pvec
                        tbb[pl.ds(ebase + jg * 16, 16)] = v

                out_start(t, b)

        for t in range(_T - _NBUF, _T):
            out_wait(t, t % _NBUF)

    return k(xT, table, pos)


def kernel(x, input_table, pos_table, positions):
    pos = jnp.take(pos_table, positions, axis=0)
    out5 = _sc_embed(x.T.astype(jnp.int32), input_table, pos)
    o = out5.reshape(_T, _EG, _B // _BB, 8, _BB)
    return o.transpose(2, 4, 0, 1, 3).reshape(_B, _T, _EMB)


# speedup vs baseline: 1.7227x; 1.7227x over previous
import functools

import jax
import jax.numpy as jnp
from jax import lax
from jax.experimental import pallas as pl
from jax.experimental.pallas import tpu as pltpu
from jax.experimental.pallas import tpu_sc as plsc

_B, _T, _EMB = 4096, 200, 64
_BB = 128
_EG = _EMB // 8
_NBUF = 4
_LOOK = 2


def _sc_embed(xT, table, pos):
    info = plsc.get_sparse_core_info()
    nw = info.num_cores * info.num_subcores
    nb = _B // _BB
    assert nb == nw

    mesh = plsc.VectorSubcoreMesh(core_axis_name="c", subcore_axis_name="s")

    @functools.partial(
        pl.kernel,
        out_type=jax.ShapeDtypeStruct((_T, _EG, nb, 8, _BB), jnp.float32),
        mesh=mesh,
        scratch_types=[
            pltpu.VMEM((_T, _BB), jnp.int32),
            pltpu.VMEM((_T, _EMB), jnp.float32),
            pltpu.VMEM((_NBUF, _BB, _EMB), jnp.float32),
            pltpu.VMEM((_NBUF, _EMB, _BB + 4), jnp.float32),
        ]
        + [pltpu.SemaphoreType.DMA] * (2 * _NBUF),
        compiler_params=pltpu.CompilerParams(
            use_tc_tiling_on_sc=False,
            needs_layout_passes=False,
            disable_bounds_checks=True
        ),
    )
    def k(xT_hbm, table_hbm, pos_hbm, out_hbm, idx_v, pos_v, rows_v, tb_v, *sems):
        sem_g = sems[:_NBUF]
        sem_o = sems[_NBUF:]
        wid = lax.axis_index("s") * info.num_cores + lax.axis_index("c")

        pltpu.sync_copy(xT_hbm.at[:, pl.ds(wid * _BB, _BB)], idx_v)
        pltpu.sync_copy(pos_hbm, pos_v)

        iota = lax.iota(jnp.int32, 16)

        def gather_start(t, b):
            pltpu.async_copy(table_hbm.at[idx_v.at[t]], rows_v.at[b], sem_g[b])

        def gather_wait(t, b):
            pltpu.make_async_copy(
                table_hbm.at[idx_v.at[t]], rows_v.at[b], sem_g[b]
            ).wait()

        def out_start(t, b):
            for e in range(_EG):
                pltpu.async_copy(
                    tb_v.at[b, pl.ds(e * 8, 8), pl.ds(0, _BB)],
                    out_hbm.at[t, e, wid],
                    sem_o[b],
                )

        def out_wait(t, b):
            for e in range(_EG):
                pltpu.make_async_copy(
                    tb_v.at[b, pl.ds(e * 8, 8), pl.ds(0, _BB)],
                    out_hbm.at[t, e, wid],
                    sem_o[b],
                ).wait()

        for t in range(_LOOK):
            gather_start(t, t % _NBUF)

        @pl.loop(0, _T // _NBUF)
        def _grp(g):
            for b in range(_NBUF):
                t = g * _NBUF + b
                tt = t + _LOOK
                b2 = (b + _LOOK) % _NBUF

                @pl.when(tt < _T)
                def _issue():
                    @pl.when(tt >= _NBUF)
                    def _drain():
                        out_wait(tt - _NBUF, b2)

                    gather_start(tt, b2)

                gather_wait(t, b)

                rb = rows_v.at[b]
                tbb = tb_v.at[b]
                pss = []
                elanes = []
                for s in range(_EMB // 16):
                    pss.append(pos_v[t, pl.ds(s * 16, 16)])
                    elanes.append(iota + (s * 16))

                @plsc.parallel_loop(0, _BB, unroll=4)
                def _row(i):
                    i_splat = jnp.full((16,), i, jnp.int32)
                    for s in range(_EMB // 16):
                        v = rb[i, pl.ds(s * 16, 16)] + pss[s]
                        plsc.store_scatter(tbb, [elanes[s], i_splat], v)

                out_start(t, b)

        for t in range(_T - _NBUF, _T):
            out_wait(t, t % _NBUF)

    return k(xT, table, pos)


def kernel(x, input_table, pos_table, positions):
    pos = jnp.take(pos_table, positions, axis=0)
    out5 = _sc_embed(x.T.astype(jnp.int32), input_table, pos)
    return out5.transpose(2, 4, 0, 1, 3).reshape(_B, _T, _EMB)
